# Initial kernel scaffold; baseline (speedup 1.0000x reference)
import jax
import jax.numpy as jnp
from jax.experimental import pallas as pl


def kernel(x, edge_index, edge_weight, enc1_W, enc2_W, z_W, z_b, dec1_W, dec1_b, dec2_W, dec2_b, xbar_W, xbar_b, mlp_W, mlp_b, m_enc1_W, m_enc2_W, m_z_W, m_z_b):
    raise NotImplementedError("write your pallas kernel here")


# speedup vs baseline: 2.4981x; 2.4981x over previous
import functools

import jax
import jax.numpy as jnp
from jax import lax
from jax.experimental import pallas as pl
from jax.experimental.pallas import tpu as pltpu
from jax.experimental.pallas import tpu_sc as plsc

N = 10000
D = 128
H = 256
Z = 64
K = 10
FC = 128
NCHUNK = 4
MOM = 0.9


_NBLK = 5
_BN = N // _NBLK


def _layer1_body(x_ref, e1_ref, me1_ref, out_ref):
    u = MOM * me1_ref[...] + (1.0 - MOM) * e1_ref[...]
    se = jnp.dot(x_ref[...], e1_ref[...], preferred_element_type=jnp.float32)
    sm = jnp.dot(x_ref[...], u, preferred_element_type=jnp.float32)
    out_ref[0] = se[:, :FC]
    out_ref[1] = se[:, FC:]
    out_ref[2] = sm[:, :FC]
    out_ref[3] = sm[:, FC:]


def _tc_layer1(x, e1, me1):
    return pl.pallas_call(
        _layer1_body,
        grid=(_NBLK,),
        in_specs=[
            pl.BlockSpec((_BN, D), lambda i: (i, 0)),
            pl.BlockSpec((D, H), lambda i: (0, 0)),
            pl.BlockSpec((D, H), lambda i: (0, 0)),
        ],
        out_specs=pl.BlockSpec((NCHUNK, _BN, FC), lambda i: (0, i, 0)),
        out_shape=jax.ShapeDtypeStruct((NCHUNK, N, FC), jnp.float32),
    )(x, e1, me1)


def _layer2_body(s1_ref, e2_ref, me2_ref, out_ref):
    h1 = jnp.maximum(jnp.concatenate([s1_ref[0], s1_ref[1]], axis=1), 0.0)
    m1 = jnp.maximum(jnp.concatenate([s1_ref[2], s1_ref[3]], axis=1), 0.0)
    u = MOM * me2_ref[...] + (1.0 - MOM) * e2_ref[...]
    se = jnp.dot(h1, e2_ref[...], preferred_element_type=jnp.float32)
    sm = jnp.dot(m1, u, preferred_element_type=jnp.float32)
    out_ref[0] = se[:, :FC]
    out_ref[1] = se[:, FC:]
    out_ref[2] = sm[:, :FC]
    out_ref[3] = sm[:, FC:]


def _tc_layer2(s1cm, e2, me2):
    return pl.pallas_call(
        _layer2_body,
        grid=(_NBLK,),
        in_specs=[
            pl.BlockSpec((NCHUNK, _BN, FC), lambda i: (0, i, 0)),
            pl.BlockSpec((H, H), lambda i: (0, 0)),
            pl.BlockSpec((H, H), lambda i: (0, 0)),
        ],
        out_specs=pl.BlockSpec((NCHUNK, _BN, FC), lambda i: (0, i, 0)),
        out_shape=jax.ShapeDtypeStruct((NCHUNK, N, FC), jnp.float32),
    )(s1cm, e2, me2)


def _head_body(s2_ref, zW_ref, zb_ref, d1W_ref, d1b_ref, d2W_ref, d2b_ref,
               xbW_ref, xbb_ref, mW_ref, mb_ref, mzW_ref, mzb_ref,
               xbar_ref, q_ref, z_ref, zm_ref):
    h2 = jnp.maximum(jnp.concatenate([s2_ref[0], s2_ref[1]], axis=1), 0.0)
    mh2 = jnp.maximum(jnp.concatenate([s2_ref[2], s2_ref[3]], axis=1), 0.0)
    z = jnp.dot(h2, zW_ref[...], preferred_element_type=jnp.float32) + zb_ref[...]
    d1 = jnp.maximum(
        jnp.dot(z, d1W_ref[...], preferred_element_type=jnp.float32) + d1b_ref[...], 0.0)
    d2 = jnp.maximum(
        jnp.dot(d1, d2W_ref[...], preferred_element_type=jnp.float32) + d2b_ref[...], 0.0)
    xbar = jnp.dot(d2, xbW_ref[...], preferred_element_type=jnp.float32) + xbb_ref[...]
    logits = jnp.dot(z, mW_ref[...], preferred_element_type=jnp.float32) + mb_ref[...]
    lmax = jnp.max(logits, axis=-1, keepdims=True)
    le = jnp.exp(logits - lmax)
    q = le / jnp.sum(le, axis=-1, keepdims=True)
    u_zW = MOM * mzW_ref[...] + (1.0 - MOM) * zW_ref[...]
    u_zb = MOM * mzb_ref[...] + (1.0 - MOM) * zb_ref[...]
    zm = jnp.dot(mh2, u_zW, preferred_element_type=jnp.float32) + u_zb
    xbar_ref[...] = xbar
    q_ref[...] = q
    z_ref[...] = z
    zm_ref[...] = zm


def _tc_head(s2cm, zW, zb, d1W, d1b, d2W, d2b, xbW, xbb, mW, mb, mzW, mzb):
    full = lambda r, c: pl.BlockSpec((r, c), lambda i: (0, 0))
    return pl.pallas_call(
        _head_body,
        grid=(_NBLK,),
        in_specs=[
            pl.BlockSpec((NCHUNK, _BN, FC), lambda i: (0, i, 0)),
            full(H, Z), full(1, Z),
            full(Z, H), full(1, H),
            full(H, H), full(1, H),
            full(H, D), full(1, D),
            full(Z, K), full(1, K),
            full(H, Z), full(1, Z),
        ],
        out_specs=[
            pl.BlockSpec((_BN, D), lambda i: (i, 0)),
            pl.BlockSpec((_BN, K), lambda i: (i, 0)),
            pl.BlockSpec((_BN, Z), lambda i: (i, 0)),
            pl.BlockSpec((_BN, Z), lambda i: (i, 0)),
        ],
        out_shape=[
            jax.ShapeDtypeStruct((N, D), jnp.float32),
            jax.ShapeDtypeStruct((N, K), jnp.float32),
            jax.ShapeDtypeStruct((N, Z), jnp.float32),
            jax.ShapeDtypeStruct((N, Z), jnp.float32),
        ],
    )(s2cm, zW, zb, d1W, d1b, d2W, d2b, xbW, xbb, mW, mb, mzW, mzb)


_GDN = lax.GatherDimensionNumbers(
    offset_dims=(), collapsed_slice_dims=(0,), start_index_map=(0,))

_EB = 80
_ZROWS = 48
_RPT = 624


def _make_spmm(E, n_subcores):
    per_tile = E // n_subcores
    n_blocks = per_tile // _EB
    assert per_tile % _EB == 0

    mesh = plsc.VectorSubcoreMesh(core_axis_name="c", subcore_axis_name="s")

    @functools.partial(
        pl.kernel,
        mesh=mesh,
        out_type=jax.ShapeDtypeStruct((NCHUNK * N, FC), jnp.float32),
        scratch_types=[
            pltpu.VMEM((_EB,), jnp.int32),
            pltpu.VMEM((_EB,), jnp.int32),
            pltpu.VMEM((_EB,), jnp.float32),
            pltpu.VMEM((_EB,), jnp.int32),
            pltpu.VMEM((_EB, FC), jnp.float32),
            pltpu.VMEM((_ZROWS, FC), jnp.float32),
            pltpu.VMEM_SHARED((N, FC), jnp.float32),
            pltpu.SemaphoreType.DMA,
        ],
    )
    def spmm(sup_hbm, src_hbm, dst_hbm, ew_hbm, out_hbm,
             srcv, dstv, ewv, gidxv, rowsv, zerov, acc, sem):
        c = lax.axis_index("c")
        s = lax.axis_index("s")
        ebase = s * per_tile

        def _zrow(r, _):
            for f in range(FC // 16):
                zerov[r, pl.ds(f * 16, 16)] = jnp.zeros((16,), jnp.float32)
            return _
        lax.fori_loop(0, _ZROWS, _zrow, None)

        last = s == n_subcores - 1
        for k in range(2):
            ch = 2 * c + k
            for j in range(_RPT // _ZROWS):
                pltpu.sync_copy(
                    zerov, acc.at[pl.ds(s * _RPT + j * _ZROWS, _ZROWS)])
            @pl.when(last)
            def _():
                pltpu.sync_copy(zerov.at[pl.ds(0, 16)],
                                acc.at[pl.ds(n_subcores * _RPT, 16)])
            plsc.subcore_barrier()

            def _block(ib, _):
                off = ebase + ib * _EB
                pltpu.sync_copy(src_hbm.at[pl.ds(off, _EB)], srcv)
                pltpu.sync_copy(dst_hbm.at[pl.ds(off, _EB)], dstv)
                pltpu.sync_copy(ew_hbm.at[pl.ds(off, _EB)], ewv)
                for g in range(_EB // 16):
                    sl = pl.ds(g * 16, 16)
                    gidxv[sl] = srcv[sl] + ch * N
                pltpu.async_copy(sup_hbm.at[gidxv], rowsv, sem).wait()
                for g in range(_EB // 16):
                    ewg = ewv[pl.ds(g * 16, 16)]
                    for j in range(16):
                        e = g * 16 + j
                        bc = lax.gather(
                            ewg, jnp.full((16, 1), j, jnp.int32),
                            _GDN, slice_sizes=(1,),
                            mode=lax.GatherScatterMode.PROMISE_IN_BOUNDS)
                        for f in range(FC // 16):
                            sl = pl.ds(f * 16, 16)
                            rowsv[e, sl] = rowsv[e, sl] * bc
                pltpu.sync_copy(rowsv, acc.at[dstv], add=True)
                return _
            lax.fori_loop(0, n_blocks, _block, None)

            plsc.subcore_barrier()
            pltpu.sync_copy(
                acc.at[pl.ds(s * _RPT, _RPT)],
                out_hbm.at[pl.ds(ch * N + s * _RPT, _RPT)])
            @pl.when(last)
            def _():
                pltpu.sync_copy(
                    acc.at[pl.ds(n_subcores * _RPT, 16)],
                    out_hbm.at[pl.ds(ch * N + n_subcores * _RPT, 16)])

    return spmm


def kernel(x, edge_index, edge_weight, enc1_W, enc2_W, z_W, z_b,
           dec1_W, dec1_b, dec2_W, dec2_b, xbar_W, xbar_b,
           mlp_W, mlp_b, m_enc1_W, m_enc2_W, m_z_W, m_z_b):
    E = edge_index.shape[1]
    src = edge_index[0]
    dst = edge_index[1]
    spmm = _make_spmm(E, 16)

    sup1 = _tc_layer1(x, enc1_W, m_enc1_W)
    s1 = spmm(sup1.reshape(NCHUNK * N, FC), src, dst, edge_weight)
    sup2 = _tc_layer2(s1.reshape(NCHUNK, N, FC), enc2_W, m_enc2_W)
    s2 = spmm(sup2.reshape(NCHUNK * N, FC), src, dst, edge_weight)
    x_bar, q, z, z_momt = _tc_head(
        s2.reshape(NCHUNK, N, FC),
        z_W, z_b.reshape(1, Z),
        dec1_W, dec1_b.reshape(1, H),
        dec2_W, dec2_b.reshape(1, H),
        xbar_W, xbar_b.reshape(1, D),
        mlp_W, mlp_b.reshape(1, K),
        m_z_W, m_z_b.reshape(1, Z))
    return (x_bar, q, z, z_momt)


# speedup vs baseline: 5.1275x; 2.0525x over previous
import functools

import jax
import jax.numpy as jnp
from jax import lax
from jax.experimental import pallas as pl
from jax.experimental.pallas import tpu as pltpu
from jax.experimental.pallas import tpu_sc as plsc

N = 10000
D = 128
H = 256
Z = 64
K = 10
FC = 128
NCHUNK = 4
MOM = 0.9


_NBLK = 5
_BN = N // _NBLK


def _layer1_body(x_ref, e1_ref, me1_ref, out_ref):
    u = MOM * me1_ref[...] + (1.0 - MOM) * e1_ref[...]
    se = jnp.dot(x_ref[...], e1_ref[...], preferred_element_type=jnp.float32)
    sm = jnp.dot(x_ref[...], u, preferred_element_type=jnp.float32)
    out_ref[0] = se[:, :FC]
    out_ref[1] = se[:, FC:]
    out_ref[2] = sm[:, :FC]
    out_ref[3] = sm[:, FC:]


def _tc_layer1(x, e1, me1):
    return pl.pallas_call(
        _layer1_body,
        grid=(_NBLK,),
        in_specs=[
            pl.BlockSpec((_BN, D), lambda i: (i, 0)),
            pl.BlockSpec((D, H), lambda i: (0, 0)),
            pl.BlockSpec((D, H), lambda i: (0, 0)),
        ],
        out_specs=pl.BlockSpec((NCHUNK, _BN, FC), lambda i: (0, i, 0)),
        out_shape=jax.ShapeDtypeStruct((NCHUNK, N, FC), jnp.float32),
    )(x, e1, me1)


def _layer2_body(s1_ref, e2_ref, me2_ref, out_ref):
    h1 = jnp.maximum(jnp.concatenate([s1_ref[0], s1_ref[1]], axis=1), 0.0)
    m1 = jnp.maximum(jnp.concatenate([s1_ref[2], s1_ref[3]], axis=1), 0.0)
    u = MOM * me2_ref[...] + (1.0 - MOM) * e2_ref[...]
    se = jnp.dot(h1, e2_ref[...], preferred_element_type=jnp.float32)
    sm = jnp.dot(m1, u, preferred_element_type=jnp.float32)
    out_ref[0] = se[:, :FC]
    out_ref[1] = se[:, FC:]
    out_ref[2] = sm[:, :FC]
    out_ref[3] = sm[:, FC:]


def _tc_layer2(s1cm, e2, me2):
    return pl.pallas_call(
        _layer2_body,
        grid=(_NBLK,),
        in_specs=[
            pl.BlockSpec((NCHUNK, _BN, FC), lambda i: (0, i, 0)),
            pl.BlockSpec((H, H), lambda i: (0, 0)),
            pl.BlockSpec((H, H), lambda i: (0, 0)),
        ],
        out_specs=pl.BlockSpec((NCHUNK, _BN, FC), lambda i: (0, i, 0)),
        out_shape=jax.ShapeDtypeStruct((NCHUNK, N, FC), jnp.float32),
    )(s1cm, e2, me2)


def _head_body(s2_ref, zW_ref, zb_ref, d1W_ref, d1b_ref, d2W_ref, d2b_ref,
               xbW_ref, xbb_ref, mW_ref, mb_ref, mzW_ref, mzb_ref,
               xbar_ref, q_ref, z_ref, zm_ref):
    h2 = jnp.maximum(jnp.concatenate([s2_ref[0], s2_ref[1]], axis=1), 0.0)
    mh2 = jnp.maximum(jnp.concatenate([s2_ref[2], s2_ref[3]], axis=1), 0.0)
    z = jnp.dot(h2, zW_ref[...], preferred_element_type=jnp.float32) + zb_ref[...]
    d1 = jnp.maximum(
        jnp.dot(z, d1W_ref[...], preferred_element_type=jnp.float32) + d1b_ref[...], 0.0)
    d2 = jnp.maximum(
        jnp.dot(d1, d2W_ref[...], preferred_element_type=jnp.float32) + d2b_ref[...], 0.0)
    xbar = jnp.dot(d2, xbW_ref[...], preferred_element_type=jnp.float32) + xbb_ref[...]
    logits = jnp.dot(z, mW_ref[...], preferred_element_type=jnp.float32) + mb_ref[...]
    lmax = jnp.max(logits, axis=-1, keepdims=True)
    le = jnp.exp(logits - lmax)
    q = le / jnp.sum(le, axis=-1, keepdims=True)
    u_zW = MOM * mzW_ref[...] + (1.0 - MOM) * zW_ref[...]
    u_zb = MOM * mzb_ref[...] + (1.0 - MOM) * zb_ref[...]
    zm = jnp.dot(mh2, u_zW, preferred_element_type=jnp.float32) + u_zb
    xbar_ref[...] = xbar
    q_ref[...] = q
    z_ref[...] = z
    zm_ref[...] = zm


def _tc_head(s2cm, zW, zb, d1W, d1b, d2W, d2b, xbW, xbb, mW, mb, mzW, mzb):
    full = lambda r, c: pl.BlockSpec((r, c), lambda i: (0, 0))
    return pl.pallas_call(
        _head_body,
        grid=(_NBLK,),
        in_specs=[
            pl.BlockSpec((NCHUNK, _BN, FC), lambda i: (0, i, 0)),
            full(H, Z), full(1, Z),
            full(Z, H), full(1, H),
            full(H, H), full(1, H),
            full(H, D), full(1, D),
            full(Z, K), full(1, K),
            full(H, Z), full(1, Z),
        ],
        out_specs=[
            pl.BlockSpec((_BN, D), lambda i: (i, 0)),
            pl.BlockSpec((_BN, K), lambda i: (i, 0)),
            pl.BlockSpec((_BN, Z), lambda i: (i, 0)),
            pl.BlockSpec((_BN, Z), lambda i: (i, 0)),
        ],
        out_shape=[
            jax.ShapeDtypeStruct((N, D), jnp.float32),
            jax.ShapeDtypeStruct((N, K), jnp.float32),
            jax.ShapeDtypeStruct((N, Z), jnp.float32),
            jax.ShapeDtypeStruct((N, Z), jnp.float32),
        ],
    )(s2cm, zW, zb, d1W, d1b, d2W, d2b, xbW, xbb, mW, mb, mzW, mzb)


_GDN = lax.GatherDimensionNumbers(
    offset_dims=(), collapsed_slice_dims=(0,), start_index_map=(0,))

_EB = 80
_ZROWS = 48
_RPT = 624


_SB = 4000


def _make_spmm(E, n_subcores):
    per_tile = E // n_subcores
    n_supers = per_tile // _SB
    n_pairs = _SB // (2 * _EB)
    assert per_tile % _SB == 0 and _SB % (2 * _EB) == 0

    mesh = plsc.VectorSubcoreMesh(core_axis_name="c", subcore_axis_name="s")

    @functools.partial(
        pl.kernel,
        mesh=mesh,
        out_type=jax.ShapeDtypeStruct((NCHUNK * N, FC), jnp.float32),
        scratch_types=[
            pltpu.VMEM((_SB,), jnp.int32),
            pltpu.VMEM((_SB,), jnp.int32),
            pltpu.VMEM((_SB,), jnp.float32),
            pltpu.VMEM((_EB,), jnp.int32),
            pltpu.VMEM((_EB,), jnp.int32),
            pltpu.VMEM((_EB,), jnp.int32),
            pltpu.VMEM((_EB, FC), jnp.float32),
            pltpu.VMEM((_EB, FC), jnp.float32),
            pltpu.VMEM((_ZROWS, FC), jnp.float32),
            pltpu.VMEM_SHARED((N, FC), jnp.float32),
            pltpu.SemaphoreType.DMA,
            pltpu.SemaphoreType.DMA,
        ],
    )
    def spmm(sup_hbm, src_hbm, dst_hbm, ew_hbm, out_hbm,
             srcv, dstv, ewv, gidxA, gidxB, dsel, rowsA, rowsB,
             zerov, acc, gsemA, gsemB):
        c = lax.axis_index("c")
        s = lax.axis_index("s")
        ebase = s * per_tile

        def _zrow(r, _):
            for f in range(FC // 16):
                zerov[r, pl.ds(f * 16, 16)] = jnp.zeros((16,), jnp.float32)
            return _
        lax.fori_loop(0, _ZROWS, _zrow, None)

        def _prep_gather(off, gidx, rows, gsem, ch):
            for g in range(_EB // 16):
                sl = pl.ds(g * 16, 16)
                gidx[sl] = srcv[pl.ds(off + g * 16, 16)] + ch * N
            pltpu.make_async_copy(sup_hbm.at[gidx], rows, gsem).start()

        def _scale_scatter(off, rows):
            for g in range(_EB // 16):
                ewg = ewv[pl.ds(off + g * 16, 16)]
                for j in range(16):
                    e = g * 16 + j
                    bc = lax.gather(
                        ewg, jnp.full((16, 1), j, jnp.int32),
                        _GDN, slice_sizes=(1,),
                        mode=lax.GatherScatterMode.PROMISE_IN_BOUNDS)
                    for f in range(FC // 16):
                        sl = pl.ds(f * 16, 16)
                        rows[e, sl] = rows[e, sl] * bc
            for g in range(_EB // 16):
                sl = pl.ds(g * 16, 16)
                dsel[sl] = dstv[pl.ds(off + g * 16, 16)]
            pltpu.sync_copy(rows, acc.at[dsel], add=True)

        last = s == n_subcores - 1

        def _chunk(k, _):
            ch = 2 * c + k
            for j in range(_RPT // _ZROWS):
                pltpu.sync_copy(
                    zerov, acc.at[pl.ds(s * _RPT + j * _ZROWS, _ZROWS)])
            @pl.when(last)
            def _():
                pltpu.sync_copy(zerov.at[pl.ds(0, 16)],
                                acc.at[pl.ds(n_subcores * _RPT, 16)])
            plsc.subcore_barrier()

            def _super(s5, _):
                soff = ebase + s5 * _SB
                pltpu.sync_copy(src_hbm.at[pl.ds(soff, _SB)], srcv)
                pltpu.sync_copy(dst_hbm.at[pl.ds(soff, _SB)], dstv)
                pltpu.sync_copy(ew_hbm.at[pl.ds(soff, _SB)], ewv)
                _prep_gather(0, gidxA, rowsA, gsemA, ch)

                def _pair(i2, _):
                    off0 = i2 * (2 * _EB)
                    off1 = off0 + _EB
                    _prep_gather(off1, gidxB, rowsB, gsemB, ch)
                    pltpu.make_async_copy(
                        sup_hbm.at[gidxA], rowsA, gsemA).wait()
                    _scale_scatter(off0, rowsA)
                    @pl.when(i2 < n_pairs - 1)
                    def _():
                        _prep_gather(off0 + 2 * _EB, gidxA, rowsA, gsemA, ch)
                    pltpu.make_async_copy(
                        sup_hbm.at[gidxB], rowsB, gsemB).wait()
                    _scale_scatter(off1, rowsB)
                    return _
                lax.fori_loop(0, n_pairs, _pair, None)
                return _
            lax.fori_loop(0, n_supers, _super, None)

            plsc.subcore_barrier()
            pltpu.sync_copy(
                acc.at[pl.ds(s * _RPT, _RPT)],
                out_hbm.at[pl.ds(ch * N + s * _RPT, _RPT)])
            @pl.when(last)
            def _():
                pltpu.sync_copy(
                    acc.at[pl.ds(n_subcores * _RPT, 16)],
                    out_hbm.at[pl.ds(ch * N + n_subcores * _RPT, 16)])
            return _

        lax.fori_loop(0, 2, _chunk, None)

    return spmm


def kernel(x, edge_index, edge_weight, enc1_W, enc2_W, z_W, z_b,
           dec1_W, dec1_b, dec2_W, dec2_b, xbar_W, xbar_b,
           mlp_W, mlp_b, m_enc1_W, m_enc2_W, m_z_W, m_z_b):
    E = edge_index.shape[1]
    src = edge_index[0]
    dst = edge_index[1]
    spmm = _make_spmm(E, 16)

    sup1 = _tc_layer1(x, enc1_W, m_enc1_W)
    s1 = spmm(sup1.reshape(NCHUNK * N, FC), src, dst, edge_weight)
    sup2 = _tc_layer2(s1.reshape(NCHUNK, N, FC), enc2_W, m_enc2_W)
    s2 = spmm(sup2.reshape(NCHUNK * N, FC), src, dst, edge_weight)
    x_bar, q, z, z_momt = _tc_head(
        s2.reshape(NCHUNK, N, FC),
        z_W, z_b.reshape(1, Z),
        dec1_W, dec1_b.reshape(1, H),
        dec2_W, dec2_b.reshape(1, H),
        xbar_W, xbar_b.reshape(1, D),
        mlp_W, mlp_b.reshape(1, K),
        m_z_W, m_z_b.reshape(1, Z))
    return (x_bar, q, z, z_momt)


# speedup vs baseline: 7.5895x; 1.4802x over previous
import functools

import jax
import jax.numpy as jnp
from jax import lax
from jax.experimental import pallas as pl
from jax.experimental.pallas import tpu as pltpu
from jax.experimental.pallas import tpu_sc as plsc

N = 10000
D = 128
H = 256
Z = 64
K = 10
FC = 128
NCHUNK = 4
MOM = 0.9


_NBLK = 5
_BN = N // _NBLK


def _layer1_body(x_ref, e1_ref, me1_ref, out_ref):
    u = MOM * me1_ref[...] + (1.0 - MOM) * e1_ref[...]
    se = jnp.dot(x_ref[...], e1_ref[...], preferred_element_type=jnp.float32)
    sm = jnp.dot(x_ref[...], u, preferred_element_type=jnp.float32)
    out_ref[0] = se[:, :FC]
    out_ref[1] = se[:, FC:]
    out_ref[2] = sm[:, :FC]
    out_ref[3] = sm[:, FC:]


def _tc_layer1(x, e1, me1):
    return pl.pallas_call(
        _layer1_body,
        grid=(_NBLK,),
        in_specs=[
            pl.BlockSpec((_BN, D), lambda i: (i, 0)),
            pl.BlockSpec((D, H), lambda i: (0, 0)),
            pl.BlockSpec((D, H), lambda i: (0, 0)),
        ],
        out_specs=pl.BlockSpec((NCHUNK, _BN, FC), lambda i: (0, i, 0)),
        out_shape=jax.ShapeDtypeStruct((NCHUNK, N, FC), jnp.float32),
    )(x, e1, me1)


def _layer2_body(s1_ref, e2_ref, me2_ref, out_ref):
    h1 = jnp.maximum(jnp.concatenate([s1_ref[0], s1_ref[1]], axis=1), 0.0)
    m1 = jnp.maximum(jnp.concatenate([s1_ref[2], s1_ref[3]], axis=1), 0.0)
    u = MOM * me2_ref[...] + (1.0 - MOM) * e2_ref[...]
    se = jnp.dot(h1, e2_ref[...], preferred_element_type=jnp.float32)
    sm = jnp.dot(m1, u, preferred_element_type=jnp.float32)
    out_ref[0] = se[:, :FC]
    out_ref[1] = se[:, FC:]
    out_ref[2] = sm[:, :FC]
    out_ref[3] = sm[:, FC:]


def _tc_layer2(s1cm, e2, me2):
    return pl.pallas_call(
        _layer2_body,
        grid=(_NBLK,),
        in_specs=[
            pl.BlockSpec((NCHUNK, _BN, FC), lambda i: (0, i, 0)),
            pl.BlockSpec((H, H), lambda i: (0, 0)),
            pl.BlockSpec((H, H), lambda i: (0, 0)),
        ],
        out_specs=pl.BlockSpec((NCHUNK, _BN, FC), lambda i: (0, i, 0)),
        out_shape=jax.ShapeDtypeStruct((NCHUNK, N, FC), jnp.float32),
    )(s1cm, e2, me2)


def _head_body(s2_ref, zW_ref, zb_ref, d1W_ref, d1b_ref, d2W_ref, d2b_ref,
               xbW_ref, xbb_ref, mW_ref, mb_ref, mzW_ref, mzb_ref,
               xbar_ref, q_ref, z_ref, zm_ref):
    h2 = jnp.maximum(jnp.concatenate([s2_ref[0], s2_ref[1]], axis=1), 0.0)
    mh2 = jnp.maximum(jnp.concatenate([s2_ref[2], s2_ref[3]], axis=1), 0.0)
    z = jnp.dot(h2, zW_ref[...], preferred_element_type=jnp.float32) + zb_ref[...]
    d1 = jnp.maximum(
        jnp.dot(z, d1W_ref[...], preferred_element_type=jnp.float32) + d1b_ref[...], 0.0)
    d2 = jnp.maximum(
        jnp.dot(d1, d2W_ref[...], preferred_element_type=jnp.float32) + d2b_ref[...], 0.0)
    xbar = jnp.dot(d2, xbW_ref[...], preferred_element_type=jnp.float32) + xbb_ref[...]
    logits = jnp.dot(z, mW_ref[...], preferred_element_type=jnp.float32) + mb_ref[...]
    lmax = jnp.max(logits, axis=-1, keepdims=True)
    le = jnp.exp(logits - lmax)
    q = le / jnp.sum(le, axis=-1, keepdims=True)
    u_zW = MOM * mzW_ref[...] + (1.0 - MOM) * zW_ref[...]
    u_zb = MOM * mzb_ref[...] + (1.0 - MOM) * zb_ref[...]
    zm = jnp.dot(mh2, u_zW, preferred_element_type=jnp.float32) + u_zb
    xbar_ref[...] = xbar
    q_ref[...] = q
    z_ref[...] = z
    zm_ref[...] = zm


def _tc_head(s2cm, zW, zb, d1W, d1b, d2W, d2b, xbW, xbb, mW, mb, mzW, mzb):
    full = lambda r, c: pl.BlockSpec((r, c), lambda i: (0, 0))
    return pl.pallas_call(
        _head_body,
        grid=(_NBLK,),
        in_specs=[
            pl.BlockSpec((NCHUNK, _BN, FC), lambda i: (0, i, 0)),
            full(H, Z), full(1, Z),
            full(Z, H), full(1, H),
            full(H, H), full(1, H),
            full(H, D), full(1, D),
            full(Z, K), full(1, K),
            full(H, Z), full(1, Z),
        ],
        out_specs=[
            pl.BlockSpec((_BN, D), lambda i: (i, 0)),
            pl.BlockSpec((_BN, K), lambda i: (i, 0)),
            pl.BlockSpec((_BN, Z), lambda i: (i, 0)),
            pl.BlockSpec((_BN, Z), lambda i: (i, 0)),
        ],
        out_shape=[
            jax.ShapeDtypeStruct((N, D), jnp.float32),
            jax.ShapeDtypeStruct((N, K), jnp.float32),
            jax.ShapeDtypeStruct((N, Z), jnp.float32),
            jax.ShapeDtypeStruct((N, Z), jnp.float32),
        ],
    )(s2cm, zW, zb, d1W, d1b, d2W, d2b, xbW, xbb, mW, mb, mzW, mzb)


_GDN = lax.GatherDimensionNumbers(
    offset_dims=(), collapsed_slice_dims=(0,), start_index_map=(0,))

_EB = 32
_NBUF = 5
_LOOK = 3
_ZROWS = 48
_RPT = 624
_SB = 4000


def _make_spmm(E, n_subcores):
    per_tile = E // n_subcores
    n_supers = per_tile // _SB
    n_rings = _SB // (_NBUF * _EB)
    n_blocks = _SB // _EB
    assert per_tile % _SB == 0 and _SB % (_NBUF * _EB) == 0

    mesh = plsc.VectorSubcoreMesh(core_axis_name="c", subcore_axis_name="s")

    @functools.partial(
        pl.kernel,
        mesh=mesh,
        out_type=jax.ShapeDtypeStruct((NCHUNK * N, FC), jnp.float32),
        scratch_types=[
            pltpu.VMEM((_SB,), jnp.int32),
            pltpu.VMEM((_SB,), jnp.int32),
            pltpu.VMEM((_SB,), jnp.float32),
            pltpu.VMEM((_NBUF, _EB), jnp.int32),
            pltpu.VMEM((_NBUF, _EB), jnp.int32),
            pltpu.VMEM((_NBUF, _EB, FC), jnp.float32),
            pltpu.VMEM((_ZROWS, FC), jnp.float32),
            pltpu.VMEM_SHARED((N, FC), jnp.float32),
        ] + [pltpu.SemaphoreType.DMA] * (2 * _NBUF),
    )
    def spmm(sup_hbm, src_hbm, dst_hbm, ew_hbm, out_hbm,
             srcv, dstv, ewv, gidx, dsel, rows, zerov, acc, *sems):
        gsem = sems[:_NBUF]
        ssem = sems[_NBUF:]
        c = lax.axis_index("c")
        s = lax.axis_index("s")
        ebase = s * per_tile

        def _zrow(r, _):
            for f in range(FC // 16):
                zerov[r, pl.ds(f * 16, 16)] = jnp.zeros((16,), jnp.float32)
            return _
        lax.fori_loop(0, _ZROWS, _zrow, None)

        def _prep_gather(off, b, ch):
            for g in range(_EB // 16):
                gidx[b, pl.ds(g * 16, 16)] = (
                    srcv[pl.ds(off + g * 16, 16)] + ch * N)
            pltpu.make_async_copy(
                sup_hbm.at[gidx.at[b]], rows.at[b], gsem[b]).start()

        def _scale(off, b):
            for g in range(_EB // 16):
                ewg = ewv[pl.ds(off + g * 16, 16)]
                for j in range(16):
                    e = g * 16 + j
                    bc = lax.gather(
                        ewg, jnp.full((16, 1), j, jnp.int32),
                        _GDN, slice_sizes=(1,),
                        mode=lax.GatherScatterMode.PROMISE_IN_BOUNDS)
                    for f in range(FC // 16):
                        sl = pl.ds(f * 16, 16)
                        rows[b, e, sl] = rows[b, e, sl] * bc
            for g in range(_EB // 16):
                dsel[b, pl.ds(g * 16, 16)] = dstv[pl.ds(off + g * 16, 16)]

        def _wait_gather(b):
            pltpu.make_async_copy(
                sup_hbm.at[gidx.at[b]], rows.at[b], gsem[b]).wait()

        def _start_scatter(b):
            pltpu.async_copy(
                rows.at[b], acc.at[dsel.at[b]], ssem[b], add=True)

        def _wait_scatter(b):
            pltpu.make_async_copy(
                rows.at[b], acc.at[dsel.at[b]], ssem[b]).wait()

        last = s == n_subcores - 1

        def _chunk(k, _):
            ch = 2 * c + k
            for j in range(_RPT // _ZROWS):
                pltpu.sync_copy(
                    zerov, acc.at[pl.ds(s * _RPT + j * _ZROWS, _ZROWS)])
            @pl.when(last)
            def _():
                pltpu.sync_copy(zerov.at[pl.ds(0, 16)],
                                acc.at[pl.ds(n_subcores * _RPT, 16)])
            plsc.subcore_barrier()

            def _super(s5, _):
                soff = ebase + s5 * _SB
                pltpu.sync_copy(src_hbm.at[pl.ds(soff, _SB)], srcv)
                pltpu.sync_copy(dst_hbm.at[pl.ds(soff, _SB)], dstv)
                pltpu.sync_copy(ew_hbm.at[pl.ds(soff, _SB)], ewv)
                for b in range(_LOOK):
                    _prep_gather(b * _EB, b, ch)

                def _ring(t, _):
                    base = t * _NBUF
                    for b in range(_NBUF):
                        blk = base + b
                        _wait_gather(b)
                        _scale(blk * _EB, b)
                        _start_scatter(b)
                        nb = (b + _LOOK) % _NBUF
                        @pl.when(blk + _LOOK < n_blocks)
                        def _():
                            @pl.when(blk >= _NBUF - _LOOK)
                            def _():
                                _wait_scatter(nb)
                            _prep_gather((blk + _LOOK) * _EB, nb, ch)
                    return _
                lax.fori_loop(0, n_rings, _ring, None)
                for b in range(_NBUF):
                    _wait_scatter(b)
                return _
            lax.fori_loop(0, n_supers, _super, None)

            plsc.subcore_barrier()
            pltpu.sync_copy(
                acc.at[pl.ds(s * _RPT, _RPT)],
                out_hbm.at[pl.ds(ch * N + s * _RPT, _RPT)])
            @pl.when(last)
            def _():
                pltpu.sync_copy(
                    acc.at[pl.ds(n_subcores * _RPT, 16)],
                    out_hbm.at[pl.ds(ch * N + n_subcores * _RPT, 16)])
            return _

        lax.fori_loop(0, 2, _chunk, None)

    return spmm


def kernel(x, edge_index, edge_weight, enc1_W, enc2_W, z_W, z_b,
           dec1_W, dec1_b, dec2_W, dec2_b, xbar_W, xbar_b,
           mlp_W, mlp_b, m_enc1_W, m_enc2_W, m_z_W, m_z_b):
    E = edge_index.shape[1]
    src = edge_index[0]
    dst = edge_index[1]
    spmm = _make_spmm(E, 16)

    sup1 = _tc_layer1(x, enc1_W, m_enc1_W)
    s1 = spmm(sup1.reshape(NCHUNK * N, FC), src, dst, edge_weight)
    sup2 = _tc_layer2(s1.reshape(NCHUNK, N, FC), enc2_W, m_enc2_W)
    s2 = spmm(sup2.reshape(NCHUNK * N, FC), src, dst, edge_weight)
    x_bar, q, z, z_momt = _tc_head(
        s2.reshape(NCHUNK, N, FC),
        z_W, z_b.reshape(1, Z),
        dec1_W, dec1_b.reshape(1, H),
        dec2_W, dec2_b.reshape(1, H),
        xbar_W, xbar_b.reshape(1, D),
        mlp_W, mlp_b.reshape(1, K),
        m_z_W, m_z_b.reshape(1, Z))
    return (x_bar, q, z, z_momt)
